# Initial kernel scaffold; baseline (speedup 1.0000x reference)
import jax
import jax.numpy as jnp
from jax.experimental import pallas as pl


def kernel(x_uint8, offs_h, offs_w):
    raise NotImplementedError("write your pallas kernel here")


# speedup vs baseline: 2.5118x; 2.5118x over previous
import functools
import jax
import jax.numpy as jnp
from jax import lax
from jax.experimental import pallas as pl
from jax.experimental.pallas import tpu as pltpu
from jax.experimental.pallas import tpu_sc as plsc

B, C, H, W = 1024, 3, 64, 64
PAD = 3
ROWS = C * H
NW = 32
SPW = B // NW
RC = 12582912.0


def _body(x_hbm, oh_hbm, ow_hbm, out_hbm, oh_v, ow_v, src_v, out_v, rowmap_v):
    wid = lax.axis_index("s") * 2 + lax.axis_index("c")
    base = wid * SPW
    pltpu.sync_copy(oh_hbm.at[pl.ds(base, SPW)], oh_v)
    pltpu.sync_copy(ow_hbm.at[pl.ds(base, SPW)], ow_v)
    iota = lax.iota(jnp.int32, 16)

    def reflect(i, n):
        i = jnp.where(i < 0, -i, i)
        return jnp.where(i > n - 1, 2 * (n - 1) - i, i)

    def sample_body(s, carry):
        b = base + s
        sv = jnp.full((16,), s, jnp.int32)
        oy = plsc.load_gather(oh_v, [sv])
        ox = plsc.load_gather(ow_v, [sv])
        cols = [reflect(iota + (16 * g - PAD) + ox, W) for g in range(4)]
        for grp in range(ROWS // 16):
            ch = (16 * grp) // H
            ybase = (16 * grp) % H
            ry = reflect(iota + (ybase - PAD) + oy, H)
            rowmap_v[pl.ds(16 * grp, 16)] = ch * H + ry
        pltpu.sync_copy(x_hbm.at[pl.ds(b * ROWS, ROWS)], src_v)

        def row_body(t, carry2):
            tv = jnp.full((16,), t, jnp.int32)
            rowv = plsc.load_gather(rowmap_v, [tv])
            for g in range(4):
                v = plsc.load_gather(src_v, [rowv, cols[g]])
                v = jnp.minimum(v, 255.0)
                v = jnp.maximum(v, 0.0)
                v = (v + RC) - RC
                out_v[t, pl.ds(16 * g, 16)] = v
            return carry2

        lax.fori_loop(0, ROWS, row_body, 0)
        pltpu.sync_copy(out_v, out_hbm.at[pl.ds(b * ROWS, ROWS)])
        return carry

    lax.fori_loop(0, SPW, sample_body, 0)


@jax.jit
def kernel(x_uint8, offs_h, offs_w):
    x2d = x_uint8.reshape(B * ROWS, W)
    oh = offs_h.reshape(B).astype(jnp.int32)
    ow = offs_w.reshape(B).astype(jnp.int32)
    mesh = plsc.VectorSubcoreMesh(core_axis_name="c", subcore_axis_name="s")
    run = pl.kernel(
        _body,
        mesh=mesh,
        compiler_params=pltpu.CompilerParams(needs_layout_passes=False),
        out_type=jax.ShapeDtypeStruct((B * ROWS, W), jnp.float32),
        scratch_types=[
            pltpu.VMEM((SPW,), jnp.int32),
            pltpu.VMEM((SPW,), jnp.int32),
            pltpu.VMEM((ROWS, W), jnp.float32),
            pltpu.VMEM((ROWS, W), jnp.float32),
            pltpu.VMEM((ROWS,), jnp.int32),
        ],
    )
    out = run(x2d, oh, ow)
    return out.reshape(B, C, H, W).astype(x_uint8.dtype)


# speedup vs baseline: 3.3577x; 1.3367x over previous
import functools
import jax
import jax.numpy as jnp
from jax import lax
from jax.experimental import pallas as pl
from jax.experimental.pallas import tpu as pltpu
from jax.experimental.pallas import tpu_sc as plsc

B, C, H, W = 1024, 3, 64, 64
PAD = 3
ROWS = C * H
NW = 32
SPW = B // NW
RC = 12582912.0


def _body(x_hbm, oh_hbm, ow_hbm, out_hbm, oh_v, ow_v, src_v, out_v, rowmap_v,
          gsem0, gsem1, ssem0, ssem1):
    wid = lax.axis_index("s") * 2 + lax.axis_index("c")
    base = wid * SPW
    pltpu.sync_copy(oh_hbm.at[pl.ds(base, SPW)], oh_v)
    pltpu.sync_copy(ow_hbm.at[pl.ds(base, SPW)], ow_v)
    iota = lax.iota(jnp.int32, 16)
    gsems = (gsem0, gsem1)
    ssems = (ssem0, ssem1)

    def reflect(i, n):
        i = jnp.where(i < 0, -i, i)
        return jnp.where(i > n - 1, 2 * (n - 1) - i, i)

    def start_load(s, p):
        pltpu.async_copy(x_hbm.at[pl.ds((base + s) * ROWS, ROWS)],
                         src_v.at[p], gsems[p])

    start_load(0, 0)
    start_load(1, 1)

    def pair_body(so, carry):
        for p in (0, 1):
            s = 2 * so + p
            pltpu.make_async_copy(x_hbm.at[pl.ds(0, ROWS)], src_v.at[p],
                                  gsems[p]).wait()
            @pl.when(so > 0)
            def _():
                pltpu.make_async_copy(out_v.at[p], out_hbm.at[pl.ds(0, ROWS)],
                                      ssems[p]).wait()

            sv = jnp.full((16,), s, jnp.int32)
            oy = plsc.load_gather(oh_v, [sv])
            ox = plsc.load_gather(ow_v, [sv])
            cols = [reflect(iota + (16 * g - PAD) + ox, W) for g in range(4)]
            for grp in range(ROWS // 16):
                ch = (16 * grp) // H
                ybase = (16 * grp) % H
                ry = reflect(iota + (ybase - PAD) + oy, H)
                rowmap_v[pl.ds(16 * grp, 16)] = ch * H + ry

            src_p = src_v.at[p]
            out_p = out_v.at[p]

            def block_body(q, c2):
                tq = jnp.full((16,), 16 * q, jnp.int32)
                for r in range(16):
                    t = 16 * q + r
                    tvr = tq + r
                    rowv = plsc.load_gather(rowmap_v, [tvr])
                    for g in range(4):
                        v = plsc.load_gather(src_p, [rowv, cols[g]])
                        v = jnp.minimum(v, 255.0)
                        v = jnp.maximum(v, 0.0)
                        v = (v + RC) - RC
                        out_p[t, pl.ds(16 * g, 16)] = v
                return c2

            lax.fori_loop(0, ROWS // 16, block_body, 0)
            pltpu.async_copy(out_p, out_hbm.at[pl.ds((base + s) * ROWS, ROWS)],
                             ssems[p])

            @pl.when(so < SPW // 2 - 1)
            def _():
                start_load(s + 2, p)
        return carry

    lax.fori_loop(0, SPW // 2, pair_body, 0)
    for p in (0, 1):
        pltpu.make_async_copy(out_v.at[p], out_hbm.at[pl.ds(0, ROWS)],
                              ssems[p]).wait()


@jax.jit
def kernel(x_uint8, offs_h, offs_w):
    x2d = x_uint8.reshape(B * ROWS, W)
    oh = offs_h.reshape(B).astype(jnp.int32)
    ow = offs_w.reshape(B).astype(jnp.int32)
    mesh = plsc.VectorSubcoreMesh(core_axis_name="c", subcore_axis_name="s")
    run = pl.kernel(
        _body,
        mesh=mesh,
        compiler_params=pltpu.CompilerParams(needs_layout_passes=False),
        out_type=jax.ShapeDtypeStruct((B * ROWS, W), jnp.float32),
        scratch_types=[
            pltpu.VMEM((SPW,), jnp.int32),
            pltpu.VMEM((SPW,), jnp.int32),
            pltpu.VMEM((2, ROWS, W), jnp.float32),
            pltpu.VMEM((2, ROWS, W), jnp.float32),
            pltpu.VMEM((ROWS,), jnp.int32),
            pltpu.SemaphoreType.DMA,
            pltpu.SemaphoreType.DMA,
            pltpu.SemaphoreType.DMA,
            pltpu.SemaphoreType.DMA,
        ],
    )
    out = run(x2d, oh, ow)
    return out.reshape(B, C, H, W).astype(x_uint8.dtype)


# speedup vs baseline: 5.8518x; 1.7428x over previous
import functools
import jax
import jax.numpy as jnp
from jax import lax
from jax.experimental import pallas as pl
from jax.experimental.pallas import tpu as pltpu
from jax.experimental.pallas import tpu_sc as plsc

B, C, H, W = 1024, 3, 64, 64
PAD = 3
ROWS = C * H
NW = 32
SPW = B // NW
RC = 12582912.0


def _body(x_hbm, oh_hbm, ow_hbm, out_hbm, oh_v, ow_v, src_v, out_v, rowmap_v,
          gsem0, gsem1, ssem0, ssem1):
    wid = lax.axis_index("s") * 2 + lax.axis_index("c")
    base = wid * SPW
    pltpu.sync_copy(oh_hbm.at[pl.ds(base, SPW)], oh_v)
    pltpu.sync_copy(ow_hbm.at[pl.ds(base, SPW)], ow_v)
    iota = lax.iota(jnp.int32, 16)
    gsems = (gsem0, gsem1)
    ssems = (ssem0, ssem1)

    def reflect(i, n):
        i = jnp.where(i < 0, -i, i)
        return jnp.where(i > n - 1, 2 * (n - 1) - i, i)

    def start_load(s, p):
        pltpu.async_copy(x_hbm.at[pl.ds((base + s) * ROWS, ROWS)],
                         src_v.at[p], gsems[p])

    start_load(0, 0)
    start_load(1, 1)

    def pair_body(so, carry):
        for p in (0, 1):
            s = 2 * so + p
            pltpu.make_async_copy(x_hbm.at[pl.ds(0, ROWS)], src_v.at[p],
                                  gsems[p]).wait()
            @pl.when(so > 0)
            def _():
                pltpu.make_async_copy(out_v.at[p], out_hbm.at[pl.ds(0, ROWS)],
                                      ssems[p]).wait()

            sv = jnp.full((16,), s, jnp.int32)
            oy = plsc.load_gather(oh_v, [sv])
            ox = plsc.load_gather(ow_v, [sv])
            cols = [reflect(iota + (16 * g - PAD) + ox, W) for g in range(4)]
            for grp in range(ROWS // 16):
                ch = (16 * grp) // H
                ybase = (16 * grp) % H
                ry = reflect(iota + (ybase - PAD) + oy, H)
                rowmap_v[pl.ds(16 * grp, 16)] = ch * H + ry

            src_p = src_v.at[p]
            out_p = out_v.at[p]

            @plsc.parallel_loop(0, ROWS, unroll=8)
            def _row(t):
                tvr = jnp.full((16,), t, jnp.int32)
                rowv = plsc.load_gather(rowmap_v, [tvr])
                for g in range(4):
                    v = plsc.load_gather(src_p, [rowv, cols[g]])
                    v = jnp.minimum(v, 255.0)
                    v = jnp.maximum(v, 0.0)
                    v = (v + RC) - RC
                    out_p[t, pl.ds(16 * g, 16)] = v
            pltpu.async_copy(out_p, out_hbm.at[pl.ds((base + s) * ROWS, ROWS)],
                             ssems[p])

            @pl.when(so < SPW // 2 - 1)
            def _():
                start_load(s + 2, p)
        return carry

    lax.fori_loop(0, SPW // 2, pair_body, 0)
    for p in (0, 1):
        pltpu.make_async_copy(out_v.at[p], out_hbm.at[pl.ds(0, ROWS)],
                              ssems[p]).wait()


@jax.jit
def kernel(x_uint8, offs_h, offs_w):
    x2d = x_uint8.reshape(B * ROWS, W)
    oh = offs_h.reshape(B).astype(jnp.int32)
    ow = offs_w.reshape(B).astype(jnp.int32)
    mesh = plsc.VectorSubcoreMesh(core_axis_name="c", subcore_axis_name="s")
    run = pl.kernel(
        _body,
        mesh=mesh,
        compiler_params=pltpu.CompilerParams(needs_layout_passes=False),
        out_type=jax.ShapeDtypeStruct((B * ROWS, W), jnp.float32),
        scratch_types=[
            pltpu.VMEM((SPW,), jnp.int32),
            pltpu.VMEM((SPW,), jnp.int32),
            pltpu.VMEM((2, ROWS, W), jnp.float32),
            pltpu.VMEM((2, ROWS, W), jnp.float32),
            pltpu.VMEM((ROWS,), jnp.int32),
            pltpu.SemaphoreType.DMA,
            pltpu.SemaphoreType.DMA,
            pltpu.SemaphoreType.DMA,
            pltpu.SemaphoreType.DMA,
        ],
    )
    out = run(x2d, oh, ow)
    return out.reshape(B, C, H, W).astype(x_uint8.dtype)
